# baseline (speedup 1.0000x reference)
import functools

import jax
import jax.numpy as jnp
from jax import lax
from jax.experimental import pallas as pl
from jax.experimental.pallas import tpu as pltpu
from jax.experimental.pallas import tpu_sc as plsc

B, C, S = 8, 1000, 8192
RATIO = 0.2
N_KEEP = int(S * RATIO)
S_BLK = 512
S_GRID = S // S_BLK

NUM_WORKERS = 32
CHUNK = (B * S) // NUM_WORKERS
ROWS = CHUNK // 128


def _gather_body(x_hbm, y_hbm, out_hbm, y_v, idx_v, vals_v, sem):
    wid = lax.axis_index("s") * 2 + lax.axis_index("c")
    base = wid * CHUNK
    pltpu.sync_copy(y_hbm.at[pl.ds(base, CHUNK)], y_v)
    b_off = (base // S) * (C * S)
    lane = lax.iota(jnp.int32, 16)
    for r in range(ROWS):
        for c8 in range(128 // 16):
            off = r * 128 + c8 * 16
            pos = base + off + lane
            yv = y_v[pl.ds(off, 16)]
            idx_v[r, pl.ds(c8 * 16, 16)] = b_off + (yv << 13) + (pos & (S - 1))
    copies = [pltpu.async_copy(x_hbm.at[idx_v.at[r]], vals_v.at[r], sem)
              for r in range(ROWS)]
    for cp in copies:
        cp.wait()
    pltpu.sync_copy(vals_v, out_hbm.at[wid])


def _sc_gather(x_flat, y_flat):
    mesh = plsc.VectorSubcoreMesh(core_axis_name="c", subcore_axis_name="s")
    f = functools.partial(
        pl.kernel,
        mesh=mesh,
        out_type=jax.ShapeDtypeStruct((NUM_WORKERS, ROWS, 128), jnp.float32),
        scratch_types=[
            pltpu.VMEM((CHUNK,), jnp.int32),
            pltpu.VMEM((ROWS, 128), jnp.int32),
            pltpu.VMEM((ROWS, 128), jnp.float32),
            pltpu.SemaphoreType.DMA,
        ],
    )(_gather_body)
    return f(x_flat, y_flat)


def _ce_topk_kernel(x_ref, g_ref, out_ref, l_ref):
    b = pl.program_id(0)
    sb = pl.program_id(1)

    xb = x_ref[0]
    e = jnp.exp(jnp.minimum(xb, 80.0))
    ones = jnp.ones((1, C), jnp.float32)
    ssum = lax.dot_general(ones, e, (((1,), (0,)), ((), ())),
                           preferred_element_type=jnp.float32)
    l = jnp.log(ssum) - g_ref[0]
    l_ref[pl.ds(b, 1), pl.ds(sb * S_BLK, S_BLK)] = l

    @pl.when((b == B - 1) & (sb == S_GRID - 1))
    def _epilogue():
        lv = l_ref[...]
        bits = lax.bitcast_convert_type(lv, jnp.int32)
        ordv = jnp.where(bits < 0, bits ^ jnp.int32(0x7FFFFFFF), bits)
        int_min = jnp.int32(-2147483648)
        p = jnp.sum((ordv >= 0).astype(jnp.int32), axis=1, keepdims=True)
        t0 = jnp.where(p >= N_KEEP, jnp.int32(0), int_min)

        def body(i, t):
            cand = t | (jnp.int32(1) << (30 - i))
            cnt = jnp.sum((ordv >= cand).astype(jnp.int32), axis=1,
                          keepdims=True)
            return jnp.where(cnt >= N_KEEP, cand, t)

        t = lax.fori_loop(0, 31, body, t0)
        vbits = jnp.where(t < 0, t ^ jnp.int32(0x7FFFFFFF), t)
        thr = lax.bitcast_convert_type(vbits, jnp.float32)
        gt = ordv > t
        cnt_gt = jnp.sum(gt.astype(jnp.float32), axis=1, keepdims=True)
        sum_gt = jnp.sum(jnp.where(gt, lv, 0.0), axis=1, keepdims=True)
        row_total = sum_gt + (N_KEEP - cnt_gt) * thr
        out_ref[0, 0] = jnp.sum(row_total) / (N_KEEP * B)


def kernel(x, y):
    g = _sc_gather(x.reshape(-1), y.reshape(-1).astype(jnp.int32))
    g = g.reshape(B, 1, S)
    out = pl.pallas_call(
        _ce_topk_kernel,
        grid=(B, S_GRID),
        in_specs=[
            pl.BlockSpec((1, C, S_BLK), lambda b, sb: (b, 0, sb)),
            pl.BlockSpec((1, 1, S_BLK), lambda b, sb: (b, 0, sb)),
        ],
        out_specs=pl.BlockSpec((1, 1), lambda b, sb: (0, 0),
                               memory_space=pltpu.SMEM),
        out_shape=jax.ShapeDtypeStruct((1, 1), jnp.float32),
        scratch_shapes=[pltpu.VMEM((B, S), jnp.float32)],
    )(x, g)
    return out[0, 0]


# speedup vs baseline: 2.2150x; 2.2150x over previous
import jax
import jax.numpy as jnp
from jax import lax
from jax.experimental import pallas as pl
from jax.experimental.pallas import tpu as pltpu

B, C, S = 8, 1000, 8192
RATIO = 0.2
N_KEEP = int(S * RATIO)
S_BLK = 512
S_GRID = S // S_BLK


def _ce_topk_kernel(x_ref, y_ref, out_ref, l_ref):
    b = pl.program_id(0)
    sb = pl.program_id(1)

    xb = x_ref[0]
    y_row = y_ref[0]
    e = jnp.exp(jnp.minimum(xb, 80.0))
    ones = jnp.ones((1, C), jnp.float32)
    ssum = lax.dot_general(ones, e, (((1,), (0,)), ((), ())),
                           preferred_element_type=jnp.float32)
    cids = lax.broadcasted_iota(jnp.int32, (C, S_BLK), 0)
    g = jnp.sum(jnp.where(cids == y_row, xb, 0.0), axis=0, keepdims=True)
    l = jnp.log(ssum) - g
    l_ref[pl.ds(b, 1), pl.ds(sb * S_BLK, S_BLK)] = l

    @pl.when((b == B - 1) & (sb == S_GRID - 1))
    def _epilogue():
        lv = l_ref[...]
        bits = lax.bitcast_convert_type(lv, jnp.int32)
        ordv = jnp.where(bits < 0, bits ^ jnp.int32(0x7FFFFFFF), bits)
        int_min = jnp.int32(-2147483648)
        p = jnp.sum((ordv >= 0).astype(jnp.int32), axis=1, keepdims=True)
        t0 = jnp.where(p >= N_KEEP, jnp.int32(0), int_min)

        def body(i, t):
            cand = t | (jnp.int32(1) << (30 - i))
            cnt = jnp.sum((ordv >= cand).astype(jnp.int32), axis=1,
                          keepdims=True)
            return jnp.where(cnt >= N_KEEP, cand, t)

        t = lax.fori_loop(0, 31, body, t0)
        vbits = jnp.where(t < 0, t ^ jnp.int32(0x7FFFFFFF), t)
        thr = lax.bitcast_convert_type(vbits, jnp.float32)
        gt = ordv > t
        cnt_gt = jnp.sum(gt.astype(jnp.float32), axis=1, keepdims=True)
        sum_gt = jnp.sum(jnp.where(gt, lv, 0.0), axis=1, keepdims=True)
        row_total = sum_gt + (N_KEEP - cnt_gt) * thr
        out_ref[0, 0] = jnp.sum(row_total) / (N_KEEP * B)


def kernel(x, y):
    out = pl.pallas_call(
        _ce_topk_kernel,
        grid=(B, S_GRID),
        in_specs=[
            pl.BlockSpec((1, C, S_BLK), lambda b, sb: (b, 0, sb)),
            pl.BlockSpec((1, 1, S_BLK), lambda b, sb: (b, 0, sb)),
        ],
        out_specs=pl.BlockSpec((1, 1), lambda b, sb: (0, 0),
                               memory_space=pltpu.SMEM),
        out_shape=jax.ShapeDtypeStruct((1, 1), jnp.float32),
        scratch_shapes=[pltpu.VMEM((B, S), jnp.float32)],
    )(x, y.reshape(B, 1, S).astype(jnp.int32))
    return out[0, 0]


# speedup vs baseline: 3.3623x; 1.5180x over previous
import jax
import jax.numpy as jnp
from jax import lax
from jax.experimental import pallas as pl
from jax.experimental.pallas import tpu as pltpu

B, C, S = 8, 1000, 8192
RATIO = 0.2
N_KEEP = int(S * RATIO)
S_BLK = 2048
S_GRID = S // S_BLK


def _ce_topk_kernel(x_ref, y_ref, out_ref, l_ref):
    b = pl.program_id(0)
    sb = pl.program_id(1)

    xb = x_ref[0]
    y_row = y_ref[0]
    e = jnp.exp(jnp.minimum(xb, 80.0))
    ssum = jnp.sum(e, axis=0, keepdims=True)
    cids = lax.broadcasted_iota(jnp.int32, (C, S_BLK), 0)
    g = jnp.sum(jnp.where(cids == y_row, xb, 0.0), axis=0, keepdims=True)
    l = jnp.log(ssum) - g
    l_ref[pl.ds(b, 1), pl.ds(sb * S_BLK, S_BLK)] = l

    @pl.when((b == B - 1) & (sb == S_GRID - 1))
    def _epilogue():
        lv = l_ref[...]
        bits = lax.bitcast_convert_type(lv, jnp.int32)
        ordv = jnp.where(bits < 0, bits ^ jnp.int32(0x7FFFFFFF), bits)
        int_min = jnp.int32(-2147483648)
        p = jnp.sum((ordv >= 0).astype(jnp.int32), axis=1, keepdims=True)
        t0 = jnp.where(p >= N_KEEP, jnp.int32(0), int_min)

        def body(i, t):
            cand = t | (jnp.int32(1) << (30 - i))
            cnt = jnp.sum((ordv >= cand).astype(jnp.int32), axis=1,
                          keepdims=True)
            return jnp.where(cnt >= N_KEEP, cand, t)

        t = lax.fori_loop(0, 31, body, t0)
        vbits = jnp.where(t < 0, t ^ jnp.int32(0x7FFFFFFF), t)
        thr = lax.bitcast_convert_type(vbits, jnp.float32)
        gt = ordv > t
        cnt_gt = jnp.sum(gt.astype(jnp.float32), axis=1, keepdims=True)
        sum_gt = jnp.sum(jnp.where(gt, lv, 0.0), axis=1, keepdims=True)
        row_total = sum_gt + (N_KEEP - cnt_gt) * thr
        out_ref[0, 0] = jnp.sum(row_total) / (N_KEEP * B)


def kernel(x, y):
    out = pl.pallas_call(
        _ce_topk_kernel,
        grid=(B, S_GRID),
        in_specs=[
            pl.BlockSpec((1, C, S_BLK), lambda b, sb: (b, 0, sb)),
            pl.BlockSpec((1, 1, S_BLK), lambda b, sb: (b, 0, sb)),
        ],
        out_specs=pl.BlockSpec((1, 1), lambda b, sb: (0, 0),
                               memory_space=pltpu.SMEM),
        out_shape=jax.ShapeDtypeStruct((1, 1), jnp.float32),
        scratch_shapes=[pltpu.VMEM((B, S), jnp.float32)],
    )(x, y.reshape(B, 1, S).astype(jnp.int32))
    return out[0, 0]


# speedup vs baseline: 3.6119x; 1.0742x over previous
import jax
import jax.numpy as jnp
from jax import lax
from jax.experimental import pallas as pl
from jax.experimental.pallas import tpu as pltpu

B, C, S = 8, 1000, 8192
RATIO = 0.2
N_KEEP = int(S * RATIO)
S_BLK = 4096
S_GRID = S // S_BLK


def _ce_topk_kernel(x_ref, y_ref, out_ref, l_ref):
    b = pl.program_id(0)
    sb = pl.program_id(1)

    xb = x_ref[0]
    y_row = y_ref[0]
    e = jnp.exp(jnp.minimum(xb, 80.0))
    ssum = jnp.sum(e, axis=0, keepdims=True)
    cids = lax.broadcasted_iota(jnp.int32, (C, S_BLK), 0)
    g = jnp.sum(jnp.where(cids == y_row, xb, 0.0), axis=0, keepdims=True)
    l = jnp.log(ssum) - g
    l_ref[pl.ds(b, 1), pl.ds(sb * S_BLK, S_BLK)] = l

    @pl.when((b == B - 1) & (sb == S_GRID - 1))
    def _epilogue():
        lv = l_ref[...]
        bits = lax.bitcast_convert_type(lv, jnp.int32)
        ordv = jnp.where(bits < 0, bits ^ jnp.int32(0x7FFFFFFF), bits)
        int_min = jnp.int32(-2147483648)
        p = jnp.sum((ordv >= 0).astype(jnp.int32), axis=1, keepdims=True)
        t0 = jnp.where(p >= N_KEEP, jnp.int32(0), int_min)

        def body(i, t):
            cand = t | (jnp.int32(1) << (30 - i))
            cnt = jnp.sum((ordv >= cand).astype(jnp.int32), axis=1,
                          keepdims=True)
            return jnp.where(cnt >= N_KEEP, cand, t)

        t = lax.fori_loop(0, 31, body, t0)
        vbits = jnp.where(t < 0, t ^ jnp.int32(0x7FFFFFFF), t)
        thr = lax.bitcast_convert_type(vbits, jnp.float32)
        gt = ordv > t
        cnt_gt = jnp.sum(gt.astype(jnp.float32), axis=1, keepdims=True)
        sum_gt = jnp.sum(jnp.where(gt, lv, 0.0), axis=1, keepdims=True)
        row_total = sum_gt + (N_KEEP - cnt_gt) * thr
        out_ref[0, 0] = jnp.sum(row_total) / (N_KEEP * B)


def kernel(x, y):
    out = pl.pallas_call(
        _ce_topk_kernel,
        grid=(B, S_GRID),
        in_specs=[
            pl.BlockSpec((1, C, S_BLK), lambda b, sb: (b, 0, sb)),
            pl.BlockSpec((1, 1, S_BLK), lambda b, sb: (b, 0, sb)),
        ],
        out_specs=pl.BlockSpec((1, 1), lambda b, sb: (0, 0),
                               memory_space=pltpu.SMEM),
        out_shape=jax.ShapeDtypeStruct((1, 1), jnp.float32),
        scratch_shapes=[pltpu.VMEM((B, S), jnp.float32)],
    )(x, y.reshape(B, 1, S).astype(jnp.int32))
    return out[0, 0]
